# baseline (device time: 335478 ns/iter reference)
import jax
import jax.numpy as jnp
from jax import lax
from jax.experimental import pallas as pl
from jax.experimental.pallas import tpu as pltpu

T = 2048
D = 4096
V_SHARD = 8192
BV = 128
NBLK = V_SHARD // BV
NS = NBLK // 2

_S, _L = 0, 1


def kernel(x, W, labels):
    labels2d = labels.reshape(T, 1)

    def body(x_ref, w_ref, lbl_ref, out_ref,
             la, lb, stats, rx, send_sem, recv_sem, ack_sem):
        j = pl.program_id(0)
        my_x = lax.axis_index("x")
        my_y = lax.axis_index("y")
        my_z = lax.axis_index("z")

        def epi(buf, blk):
            lg = buf[...]
            offset = my_x * V_SHARD + blk * BV
            idx = lbl_ref[...] - offset
            col = lax.broadcasted_iota(jnp.int32, (T, BV), 1)
            lval = jnp.sum(jnp.where(col == idx, lg, 0.0),
                           axis=1, keepdims=True)
            s = jnp.sum(jnp.exp(lg), axis=1, keepdims=True)
            return s, lval

        sA, lA = epi(la, 2 * j - 2)
        la[...] = jnp.dot(x_ref[...], w_ref[:, :BV],
                          preferred_element_type=jnp.float32)
        sB, lB = epi(lb, 2 * j - 1)
        lb[...] = jnp.dot(x_ref[...], w_ref[:, BV:],
                          preferred_element_type=jnp.float32)

        s_new = sA + sB
        l_new = lA + lB
        stats[:, _S:_S + 1] = jnp.where(j == 1, s_new,
                                        stats[:, _S:_S + 1] + s_new)
        stats[:, _L:_L + 1] = jnp.where(j == 1, l_new,
                                        stats[:, _L:_L + 1] + l_new)

        @pl.when(j == NS - 1)
        def _():
            sA2, lA2 = epi(la, 2 * j)
            sB2, lB2 = epi(lb, 2 * j + 1)
            stats[:, _S:_S + 1] = stats[:, _S:_S + 1] + sA2 + sB2
            stats[:, _L:_L + 1] = stats[:, _L:_L + 1] + lA2 + lB2

            partner = (1 - my_x, my_y, my_z)
            rdma = pltpu.make_async_remote_copy(
                src_ref=stats, dst_ref=rx,
                send_sem=send_sem, recv_sem=recv_sem,
                device_id=partner,
                device_id_type=pl.DeviceIdType.MESH)
            rdma.start()
            rdma.wait()

            s_tot = stats[:, _S:_S + 1] + rx[:, _S:_S + 1]
            l_tot = stats[:, _L:_L + 1] + rx[:, _L:_L + 1]
            out_ref[...] = jnp.log(s_tot) - l_tot

            pl.semaphore_signal(ack_sem, 1, device_id=partner,
                                device_id_type=pl.DeviceIdType.MESH)
            pl.semaphore_wait(ack_sem, 1)

    out = pl.pallas_call(
        body,
        grid=(NS,),
        in_specs=[
            pl.BlockSpec((T, D), lambda j: (0, 0)),
            pl.BlockSpec((D, 2 * BV), lambda j: (0, j)),
            pl.BlockSpec((T, 1), lambda j: (0, 0)),
        ],
        out_specs=pl.BlockSpec((T, 1), lambda j: (0, 0)),
        out_shape=jax.ShapeDtypeStruct((T, 1), jnp.float32),
        scratch_shapes=[
            pltpu.VMEM((T, BV), jnp.float32),
            pltpu.VMEM((T, BV), jnp.float32),
            pltpu.VMEM((T, 128), jnp.float32),
            pltpu.VMEM((T, 128), jnp.float32),
            pltpu.SemaphoreType.DMA,
            pltpu.SemaphoreType.DMA,
            pltpu.SemaphoreType.REGULAR,
        ],
        compiler_params=pltpu.CompilerParams(
            dimension_semantics=("arbitrary",),
            vmem_limit_bytes=64 * 1024 * 1024,
        ),
    )(x, W, labels2d)
    return out.reshape(T)


# device time: 196692 ns/iter; 1.7056x vs baseline; 1.7056x over previous
import jax
import jax.numpy as jnp
from jax import lax
from jax.experimental import pallas as pl
from jax.experimental.pallas import tpu as pltpu

T = 2048
D = 4096
V_SHARD = 8192
BV = 512
NBLK = V_SHARD // BV

_S, _L = 0, 1


def kernel(x, W, labels):
    labels2d = labels.reshape(T, 1)

    def body(x_ref, w_ref, lbl_ref, out_ref,
             stats, rx, send_sem, recv_sem, ack_sem):
        j = pl.program_id(0)
        my_x = lax.axis_index("x")
        my_y = lax.axis_index("y")
        my_z = lax.axis_index("z")

        logits = jnp.dot(x_ref[...], w_ref[...],
                         preferred_element_type=jnp.float32)

        offset = my_x * V_SHARD + j * BV
        idx = lbl_ref[...] - offset
        col = lax.broadcasted_iota(jnp.int32, (T, BV), 1)
        lval = jnp.sum(jnp.where(col == idx, logits, 0.0),
                       axis=1, keepdims=True)
        s = jnp.sum(jnp.exp(logits), axis=1, keepdims=True)

        @pl.when(j == 0)
        def _():
            stats[:, _S:_S + 1] = s
            stats[:, _L:_L + 1] = lval

        @pl.when(j > 0)
        def _():
            stats[:, _S:_S + 1] = stats[:, _S:_S + 1] + s
            stats[:, _L:_L + 1] = stats[:, _L:_L + 1] + lval

        @pl.when(j == NBLK - 1)
        def _():
            partner = (1 - my_x, my_y, my_z)
            rdma = pltpu.make_async_remote_copy(
                src_ref=stats, dst_ref=rx,
                send_sem=send_sem, recv_sem=recv_sem,
                device_id=partner,
                device_id_type=pl.DeviceIdType.MESH)
            rdma.start()
            rdma.wait()

            s_tot = stats[:, _S:_S + 1] + rx[:, _S:_S + 1]
            l_tot = stats[:, _L:_L + 1] + rx[:, _L:_L + 1]
            out_ref[...] = jnp.log(s_tot) - l_tot

            pl.semaphore_signal(ack_sem, 1, device_id=partner,
                                device_id_type=pl.DeviceIdType.MESH)
            pl.semaphore_wait(ack_sem, 1)

    out = pl.pallas_call(
        body,
        grid=(NBLK,),
        in_specs=[
            pl.BlockSpec((T, D), lambda j: (0, 0)),
            pl.BlockSpec((D, BV), lambda j: (0, j)),
            pl.BlockSpec((T, 1), lambda j: (0, 0)),
        ],
        out_specs=pl.BlockSpec((T, 1), lambda j: (0, 0)),
        out_shape=jax.ShapeDtypeStruct((T, 1), jnp.float32),
        scratch_shapes=[
            pltpu.VMEM((T, 128), jnp.float32),
            pltpu.VMEM((T, 128), jnp.float32),
            pltpu.SemaphoreType.DMA,
            pltpu.SemaphoreType.DMA,
            pltpu.SemaphoreType.REGULAR,
        ],
        compiler_params=pltpu.CompilerParams(
            dimension_semantics=("arbitrary",),
            vmem_limit_bytes=64 * 1024 * 1024,
        ),
    )(x, W, labels2d)
    return out.reshape(T)
